# baseline (speedup 1.0000x reference)
import jax
import jax.numpy as jnp
from jax import lax
from jax.experimental import pallas as pl
from jax.experimental.pallas import tpu as pltpu
from jax.experimental.pallas import tpu_sc as plsc

_N = 10000
_E = 320000
_H = 128
_C = 40

_CH = 80
_NW = 32
_SB = 25
_NSB = _E // _NW // _CH // _SB
_NP = 10240


def _sc_agg_body(h, srcm, dstm, zbig, zsmall,
                 agg_out, cnt_out,
                 idx_s, idx_d, rows0, rows1, rows2, ones, agg_sh, cnt_sh,
                 g0, g1, g2, s0, s1, s2):
    cid = lax.axis_index("c")
    sid = lax.axis_index("s")
    wid = cid * 16 + sid

    pltpu.sync_copy(zbig.at[pl.ds(sid * 640, 640)],
                    agg_sh.at[pl.ds(sid * 640, 640)])

    pltpu.sync_copy(zsmall.at[pl.ds(sid * 640, 640)],
                    cnt_sh.at[pl.ds(sid * 640, 640)])

    for i in range(_CH // 16):
        ones[pl.ds(i * 16, 16)] = jnp.ones((16,), jnp.float32)

    plsc.subcore_barrier()

    bufs = [(rows0, g0, s0), (rows1, g1, s1), (rows2, g2, s2)]

    def gather(c, k):
        pltpu.async_copy(h.at[idx_s.at[c]], bufs[k][0], bufs[k][1])

    def gwait(k):
        pltpu.make_async_copy(h.at[idx_s.at[0]], bufs[k][0], bufs[k][1]).wait()

    def sc_issue(c, k):
        pltpu.async_copy(bufs[k][0], agg_sh.at[idx_d.at[c]], bufs[k][2],
                         add=True)
        pltpu.async_copy(ones, cnt_sh.at[idx_d.at[c]], bufs[k][2], add=True)

    def swait(k):
        pltpu.make_async_copy(bufs[k][0], agg_sh.at[idx_d.at[0]],
                              bufs[k][2]).wait()
        pltpu.make_async_copy(ones, cnt_sh.at[idx_d.at[0]],
                              bufs[k][2]).wait()

    def superblock(sb, _):
        pltpu.sync_copy(srcm.at[wid].at[sb], idx_s)
        pltpu.sync_copy(dstm.at[wid].at[sb], idx_d)

        gather(0, 0)
        for c in range(_SB):
            k = c % 3
            gwait(k)
            sc_issue(c, k)
            if c + 1 < _SB:
                kn = (c + 1) % 3
                if c >= 2:
                    swait(kn)
                gather(c + 1, kn)
        for k in range(3):
            swait(k)
        return ()

    lax.fori_loop(0, _NSB, superblock, ())

    plsc.subcore_barrier()

    pltpu.sync_copy(agg_sh.at[pl.ds(sid * 640, 640)],
                    agg_out.at[cid].at[pl.ds(sid * 640, 640)])

    pltpu.sync_copy(cnt_sh.at[pl.ds(sid * 640, 640)],
                    cnt_out.at[cid].at[pl.ds(sid * 640, 640)])


_sc_aggregate = pl.kernel(
    _sc_agg_body,
    out_type=[
        jax.ShapeDtypeStruct((2, _NP, _H), jnp.float32),
        jax.ShapeDtypeStruct((2, _NP), jnp.float32),
    ],
    mesh=plsc.VectorSubcoreMesh(core_axis_name="c", subcore_axis_name="s"),
    scratch_types=[
        pltpu.VMEM((_SB, _CH), jnp.int32),
        pltpu.VMEM((_SB, _CH), jnp.int32),
        pltpu.VMEM((_CH, _H), jnp.float32),
        pltpu.VMEM((_CH, _H), jnp.float32),
        pltpu.VMEM((_CH, _H), jnp.float32),
        pltpu.VMEM((_CH,), jnp.float32),
        pltpu.VMEM_SHARED((_NP, _H), jnp.float32),
        pltpu.VMEM_SHARED((_NP,), jnp.float32),
        pltpu.SemaphoreType.DMA,
        pltpu.SemaphoreType.DMA,
        pltpu.SemaphoreType.DMA,
        pltpu.SemaphoreType.DMA,
        pltpu.SemaphoreType.DMA,
        pltpu.SemaphoreType.DMA,
    ],
)


_BN = 1000


def _embed_body(x_ref, w_ref, b_ref, o_ref):
    o_ref[...] = (jnp.dot(x_ref[...], w_ref[...],
                          preferred_element_type=jnp.float32) + b_ref[...])


def _embed(x, w, b):
    return pl.pallas_call(
        _embed_body,
        grid=(_N // _BN,),
        in_specs=[
            pl.BlockSpec((_BN, _H), lambda i: (i, 0)),
            pl.BlockSpec((_H, _H), lambda i: (0, 0)),
            pl.BlockSpec((1, _H), lambda i: (0, 0)),
        ],
        out_specs=pl.BlockSpec((_BN, _H), lambda i: (i, 0)),
        out_shape=jax.ShapeDtypeStruct((_N, _H), jnp.float32),
    )(x, w, b)


def _layer_body(p0_ref, p1_ref, c0_ref, c1_ref, h_ref, w_ref, b_ref, o_ref):
    deg = jnp.maximum(c0_ref[...] + c1_ref[...], 1.0)
    a = (p0_ref[...] + p1_ref[...]) / deg
    y = jnp.dot(a, w_ref[...], preferred_element_type=jnp.float32) + b_ref[...]
    o_ref[...] = jnp.maximum(y, 0.0) + h_ref[...]


def _layer(p0, p1, c0, c1, h, w, b):
    return pl.pallas_call(
        _layer_body,
        grid=(_N // _BN,),
        in_specs=[
            pl.BlockSpec((_BN, _H), lambda i: (i, 0)),
            pl.BlockSpec((_BN, _H), lambda i: (i, 0)),
            pl.BlockSpec((_BN, 1), lambda i: (i, 0)),
            pl.BlockSpec((_BN, 1), lambda i: (i, 0)),
            pl.BlockSpec((_BN, _H), lambda i: (i, 0)),
            pl.BlockSpec((_H, _H), lambda i: (0, 0)),
            pl.BlockSpec((1, _H), lambda i: (0, 0)),
        ],
        out_specs=pl.BlockSpec((_BN, _H), lambda i: (i, 0)),
        out_shape=jax.ShapeDtypeStruct((_N, _H), jnp.float32),
    )(p0, p1, c0, c1, h, w, b)


def _head_body(h_ref, w_ref, b_ref, o_ref):
    y = (jnp.dot(h_ref[...], w_ref[...], preferred_element_type=jnp.float32)
         + b_ref[...])
    m = jnp.max(y, axis=-1, keepdims=True)
    lse = jnp.log(jnp.sum(jnp.exp(y - m), axis=-1, keepdims=True)) + m
    o_ref[...] = y - lse


def _head(h, w, b):
    return pl.pallas_call(
        _head_body,
        grid=(_N // _BN,),
        in_specs=[
            pl.BlockSpec((_BN, _H), lambda i: (i, 0)),
            pl.BlockSpec((_H, _C), lambda i: (0, 0)),
            pl.BlockSpec((1, _C), lambda i: (0, 0)),
        ],
        out_specs=pl.BlockSpec((_BN, _C), lambda i: (i, 0)),
        out_shape=jax.ShapeDtypeStruct((_N, _C), jnp.float32),
    )(h, w, b)


@jax.jit
def kernel(x, edge_index, W_embed, b_embed, conv_W, conv_b,
           bn_gamma, bn_beta, bn_mean, bn_var, W_out, b_out):
    srcm = edge_index[0].reshape(_NW, _NSB, _SB, _CH)
    dstm = edge_index[1].reshape(_NW, _NSB, _SB, _CH)
    zbig = jnp.zeros((_NP, _H), jnp.float32)
    zsmall = jnp.zeros((_NP,), jnp.float32)

    s = bn_gamma / jnp.sqrt(bn_var + 1e-5)
    w_fold = conv_W * s[:, None, :]
    b_fold = conv_b * s + bn_beta - bn_mean * s

    h = _embed(x, W_embed, b_embed.reshape(1, _H))

    for i in range(3):
        agg, cnt = _sc_aggregate(h, srcm, dstm, zbig, zsmall)
        h = _layer(agg[0, :_N], agg[1, :_N],
                   cnt[0, :_N].reshape(_N, 1), cnt[1, :_N].reshape(_N, 1), h,
                   w_fold[i], b_fold[i].reshape(1, _H))

    return _head(h, W_out, b_out.reshape(1, _C))


# speedup vs baseline: 1.2496x; 1.2496x over previous
import functools

import jax
import jax.numpy as jnp
from jax import lax
from jax.experimental import pallas as pl
from jax.experimental.pallas import tpu as pltpu
from jax.experimental.pallas import tpu_sc as plsc

_N = 10000
_E = 320000
_H = 128
_C = 40

_CH = 80
_NW = 32
_SB = 25
_NSB = _E // _NW // _CH // _SB
_NP = 10240


def _sc_agg_body(with_cnt, h, srcm, dstm, zbig, zsmall,
                 *refs):
    if with_cnt:
        (agg_out, cnt_out,
         idx_s, idx_d, rows0, rows1, ones, agg_sh, cnt_sh, sem0, sem1) = refs
    else:
        (agg_out, idx_s, idx_d, rows0, rows1, agg_sh, sem0, sem1) = refs
    cid = lax.axis_index("c")
    sid = lax.axis_index("s")
    wid = cid * 16 + sid

    pltpu.sync_copy(zbig.at[pl.ds(sid * 640, 640)],
                    agg_sh.at[pl.ds(sid * 640, 640)])

    if with_cnt:
        pltpu.sync_copy(zsmall.at[pl.ds(sid * 640, 640)],
                        cnt_sh.at[pl.ds(sid * 640, 640)])
        for i in range(_CH // 16):
            ones[pl.ds(i * 16, 16)] = jnp.ones((16,), jnp.float32)

    plsc.subcore_barrier()

    def gather(c, rows, sem):
        pltpu.async_copy(h.at[idx_s.at[c]], rows, sem)

    def gwait(rows, sem):
        pltpu.make_async_copy(h.at[idx_s.at[0]], rows, sem).wait()

    def scatter(c, rows):
        pltpu.sync_copy(rows, agg_sh.at[idx_d.at[c]], add=True)
        if with_cnt:
            pltpu.sync_copy(ones, cnt_sh.at[idx_d.at[c]], add=True)

    def superblock(sb, _):
        pltpu.sync_copy(srcm.at[wid].at[sb], idx_s)
        pltpu.sync_copy(dstm.at[wid].at[sb], idx_d)

        gather(0, rows0, sem0)

        def pair(i, __):
            c = i * 2
            gather(c + 1, rows1, sem1)
            gwait(rows0, sem0)
            scatter(c, rows0)
            gather(c + 2, rows0, sem0)
            gwait(rows1, sem1)
            scatter(c + 1, rows1)
            return ()

        lax.fori_loop(0, (_SB - 1) // 2, pair, ())
        gwait(rows0, sem0)
        scatter(_SB - 1, rows0)
        return ()

    lax.fori_loop(0, _NSB, superblock, ())

    plsc.subcore_barrier()

    pltpu.sync_copy(agg_sh.at[pl.ds(sid * 640, 640)],
                    agg_out.at[cid].at[pl.ds(sid * 640, 640)])

    if with_cnt:
        pltpu.sync_copy(cnt_sh.at[pl.ds(sid * 640, 640)],
                        cnt_out.at[cid].at[pl.ds(sid * 640, 640)])


def _make_sc_aggregate(with_cnt):
    out_type = [jax.ShapeDtypeStruct((2, _NP, _H), jnp.float32)]
    scratch = [
        pltpu.VMEM((_SB, _CH), jnp.int32),
        pltpu.VMEM((_SB, _CH), jnp.int32),
        pltpu.VMEM((_CH, _H), jnp.float32),
        pltpu.VMEM((_CH, _H), jnp.float32),
    ]
    if with_cnt:
        out_type.append(jax.ShapeDtypeStruct((2, _NP), jnp.float32))
        scratch.append(pltpu.VMEM((_CH,), jnp.float32))
    scratch.append(pltpu.VMEM_SHARED((_NP, _H), jnp.float32))
    if with_cnt:
        scratch.append(pltpu.VMEM_SHARED((_NP,), jnp.float32))
    scratch += [pltpu.SemaphoreType.DMA, pltpu.SemaphoreType.DMA]
    return pl.kernel(
        functools.partial(_sc_agg_body, with_cnt),
        out_type=out_type,
        mesh=plsc.VectorSubcoreMesh(core_axis_name="c", subcore_axis_name="s"),
        scratch_types=scratch,
    )


_sc_aggregate_cnt = _make_sc_aggregate(True)
_sc_aggregate = _make_sc_aggregate(False)


_BN = 1000


def _embed_body(x_ref, w_ref, b_ref, o_ref):
    o_ref[...] = (jnp.dot(x_ref[...], w_ref[...],
                          preferred_element_type=jnp.float32) + b_ref[...])


def _embed(x, w, b):
    return pl.pallas_call(
        _embed_body,
        grid=(_N // _BN,),
        in_specs=[
            pl.BlockSpec((_BN, _H), lambda i: (i, 0)),
            pl.BlockSpec((_H, _H), lambda i: (0, 0)),
            pl.BlockSpec((1, _H), lambda i: (0, 0)),
        ],
        out_specs=pl.BlockSpec((_BN, _H), lambda i: (i, 0)),
        out_shape=jax.ShapeDtypeStruct((_N, _H), jnp.float32),
    )(x, w, b)


def _layer_body(p0_ref, p1_ref, c0_ref, c1_ref, h_ref, w_ref, b_ref, o_ref):
    deg = jnp.maximum(c0_ref[...] + c1_ref[...], 1.0)
    a = (p0_ref[...] + p1_ref[...]) / deg
    y = jnp.dot(a, w_ref[...], preferred_element_type=jnp.float32) + b_ref[...]
    o_ref[...] = jnp.maximum(y, 0.0) + h_ref[...]


def _layer(p0, p1, c0, c1, h, w, b):
    return pl.pallas_call(
        _layer_body,
        grid=(_N // _BN,),
        in_specs=[
            pl.BlockSpec((_BN, _H), lambda i: (i, 0)),
            pl.BlockSpec((_BN, _H), lambda i: (i, 0)),
            pl.BlockSpec((_BN, 1), lambda i: (i, 0)),
            pl.BlockSpec((_BN, 1), lambda i: (i, 0)),
            pl.BlockSpec((_BN, _H), lambda i: (i, 0)),
            pl.BlockSpec((_H, _H), lambda i: (0, 0)),
            pl.BlockSpec((1, _H), lambda i: (0, 0)),
        ],
        out_specs=pl.BlockSpec((_BN, _H), lambda i: (i, 0)),
        out_shape=jax.ShapeDtypeStruct((_N, _H), jnp.float32),
    )(p0, p1, c0, c1, h, w, b)


def _last_body(p0_ref, p1_ref, c0_ref, c1_ref, h_ref, w_ref, b_ref,
               wo_ref, bo_ref, o_ref):
    deg = jnp.maximum(c0_ref[...] + c1_ref[...], 1.0)
    a = (p0_ref[...] + p1_ref[...]) / deg
    t = jnp.dot(a, w_ref[...], preferred_element_type=jnp.float32) + b_ref[...]
    hh = jnp.maximum(t, 0.0) + h_ref[...]
    y = (jnp.dot(hh, wo_ref[...], preferred_element_type=jnp.float32)
         + bo_ref[...])
    m = jnp.max(y, axis=-1, keepdims=True)
    lse = jnp.log(jnp.sum(jnp.exp(y - m), axis=-1, keepdims=True)) + m
    o_ref[...] = y - lse


def _last(p0, p1, c0, c1, h, w, b, wo, bo):
    return pl.pallas_call(
        _last_body,
        grid=(_N // _BN,),
        in_specs=[
            pl.BlockSpec((_BN, _H), lambda i: (i, 0)),
            pl.BlockSpec((_BN, _H), lambda i: (i, 0)),
            pl.BlockSpec((_BN, 1), lambda i: (i, 0)),
            pl.BlockSpec((_BN, 1), lambda i: (i, 0)),
            pl.BlockSpec((_BN, _H), lambda i: (i, 0)),
            pl.BlockSpec((_H, _H), lambda i: (0, 0)),
            pl.BlockSpec((1, _H), lambda i: (0, 0)),
            pl.BlockSpec((_H, _C), lambda i: (0, 0)),
            pl.BlockSpec((1, _C), lambda i: (0, 0)),
        ],
        out_specs=pl.BlockSpec((_BN, _C), lambda i: (i, 0)),
        out_shape=jax.ShapeDtypeStruct((_N, _C), jnp.float32),
    )(p0, p1, c0, c1, h, w, b, wo, bo)


@jax.jit
def kernel(x, edge_index, W_embed, b_embed, conv_W, conv_b,
           bn_gamma, bn_beta, bn_mean, bn_var, W_out, b_out):
    srcm = edge_index[0].reshape(_NW, _NSB, _SB, _CH)
    dstm = edge_index[1].reshape(_NW, _NSB, _SB, _CH)
    zbig = jnp.zeros((_NP, _H), jnp.float32)
    zsmall = jnp.zeros((_NP,), jnp.float32)

    s = bn_gamma / jnp.sqrt(bn_var + 1e-5)
    w_fold = conv_W * s[:, None, :]
    b_fold = conv_b * s + bn_beta - bn_mean * s

    h = _embed(x, W_embed, b_embed.reshape(1, _H))

    c0 = c1 = None
    for i in range(3):
        if i == 0:
            agg, cnt = _sc_aggregate_cnt(h, srcm, dstm, zbig, zsmall)
            c0 = cnt[0, :_N].reshape(_N, 1)
            c1 = cnt[1, :_N].reshape(_N, 1)
        else:
            (agg,) = _sc_aggregate(h, srcm, dstm, zbig, zsmall)
        if i < 2:
            h = _layer(agg[0, :_N], agg[1, :_N], c0, c1, h,
                       w_fold[i], b_fold[i].reshape(1, _H))
        else:
            return _last(agg[0, :_N], agg[1, :_N], c0, c1, h,
                         w_fold[i], b_fold[i].reshape(1, _H),
                         W_out, b_out.reshape(1, _C))


# speedup vs baseline: 1.2912x; 1.0333x over previous
import functools

import jax
import jax.numpy as jnp
from jax import lax
from jax.experimental import pallas as pl
from jax.experimental.pallas import tpu as pltpu
from jax.experimental.pallas import tpu_sc as plsc

_N = 10000
_E = 320000
_H = 128
_C = 40

_CH = 128
_NW = 32
_SB = 13
_NSB = 6
_TAIL = 16
_NP = 10240


def _sc_agg_body(with_cnt, h, srcm, dstm, srct, dstt, zbig, zsmall,
                 *refs):
    if with_cnt:
        (agg_out, cnt_out,
         idx_s, idx_d, idx_st, idx_dt, rows0, rows1, ones, agg_sh, cnt_sh,
         sem0, sem1) = refs
    else:
        (agg_out, idx_s, idx_d, idx_st, idx_dt, rows0, rows1, agg_sh,
         sem0, sem1) = refs
    cid = lax.axis_index("c")
    sid = lax.axis_index("s")
    wid = cid * 16 + sid

    pltpu.sync_copy(zbig.at[pl.ds(sid * 640, 640)],
                    agg_sh.at[pl.ds(sid * 640, 640)])

    if with_cnt:
        pltpu.sync_copy(zsmall.at[pl.ds(sid * 640, 640)],
                        cnt_sh.at[pl.ds(sid * 640, 640)])
        for i in range(_CH // 16):
            ones[pl.ds(i * 16, 16)] = jnp.ones((16,), jnp.float32)

    plsc.subcore_barrier()

    def gather(c, rows, sem):
        pltpu.async_copy(h.at[idx_s.at[c]], rows, sem)

    def gwait(rows, sem):
        pltpu.make_async_copy(h.at[idx_s.at[0]], rows, sem).wait()

    def scatter(c, rows):
        pltpu.sync_copy(rows, agg_sh.at[idx_d.at[c]], add=True)
        if with_cnt:
            pltpu.sync_copy(ones, cnt_sh.at[idx_d.at[c]], add=True)

    def superblock(sb, _):
        pltpu.sync_copy(srcm.at[wid].at[sb], idx_s)
        pltpu.sync_copy(dstm.at[wid].at[sb], idx_d)

        gather(0, rows0, sem0)

        def pair(i, __):
            c = i * 2
            gather(c + 1, rows1, sem1)
            gwait(rows0, sem0)
            scatter(c, rows0)
            gather(c + 2, rows0, sem0)
            gwait(rows1, sem1)
            scatter(c + 1, rows1)
            return ()

        lax.fori_loop(0, (_SB - 1) // 2, pair, ())
        gwait(rows0, sem0)
        scatter(_SB - 1, rows0)
        return ()

    lax.fori_loop(0, _NSB, superblock, ())

    pltpu.sync_copy(srct.at[wid], idx_st)
    pltpu.sync_copy(dstt.at[wid], idx_dt)
    trows = rows0.at[pl.ds(0, _TAIL)]
    pltpu.async_copy(h.at[idx_st.at[0]], trows, sem0).wait()
    pltpu.sync_copy(trows, agg_sh.at[idx_dt.at[0]], add=True)
    if with_cnt:
        pltpu.sync_copy(ones.at[pl.ds(0, _TAIL)],
                        cnt_sh.at[idx_dt.at[0]], add=True)

    plsc.subcore_barrier()

    pltpu.sync_copy(agg_sh.at[pl.ds(sid * 640, 640)],
                    agg_out.at[cid].at[pl.ds(sid * 640, 640)])

    if with_cnt:
        pltpu.sync_copy(cnt_sh.at[pl.ds(sid * 640, 640)],
                        cnt_out.at[cid].at[pl.ds(sid * 640, 640)])


def _make_sc_aggregate(with_cnt):
    out_type = [jax.ShapeDtypeStruct((2, _NP, _H), jnp.float32)]
    scratch = [
        pltpu.VMEM((_SB, _CH), jnp.int32),
        pltpu.VMEM((_SB, _CH), jnp.int32),
        pltpu.VMEM((1, _TAIL), jnp.int32),
        pltpu.VMEM((1, _TAIL), jnp.int32),
        pltpu.VMEM((_CH, _H), jnp.float32),
        pltpu.VMEM((_CH, _H), jnp.float32),
    ]
    if with_cnt:
        out_type.append(jax.ShapeDtypeStruct((2, _NP), jnp.float32))
        scratch.append(pltpu.VMEM((_CH,), jnp.float32))
    scratch.append(pltpu.VMEM_SHARED((_NP, _H), jnp.float32))
    if with_cnt:
        scratch.append(pltpu.VMEM_SHARED((_NP,), jnp.float32))
    scratch += [pltpu.SemaphoreType.DMA, pltpu.SemaphoreType.DMA]
    return pl.kernel(
        functools.partial(_sc_agg_body, with_cnt),
        out_type=out_type,
        mesh=plsc.VectorSubcoreMesh(core_axis_name="c", subcore_axis_name="s"),
        scratch_types=scratch,
    )


_sc_aggregate_cnt = _make_sc_aggregate(True)
_sc_aggregate = _make_sc_aggregate(False)


_BN = 1000


def _embed_body(x_ref, w_ref, b_ref, o_ref):
    o_ref[...] = (jnp.dot(x_ref[...], w_ref[...],
                          preferred_element_type=jnp.float32) + b_ref[...])


def _embed(x, w, b):
    return pl.pallas_call(
        _embed_body,
        grid=(_N // _BN,),
        in_specs=[
            pl.BlockSpec((_BN, _H), lambda i: (i, 0)),
            pl.BlockSpec((_H, _H), lambda i: (0, 0)),
            pl.BlockSpec((1, _H), lambda i: (0, 0)),
        ],
        out_specs=pl.BlockSpec((_BN, _H), lambda i: (i, 0)),
        out_shape=jax.ShapeDtypeStruct((_N, _H), jnp.float32),
    )(x, w, b)


def _layer_body(p0_ref, p1_ref, c0_ref, c1_ref, h_ref, w_ref, b_ref, o_ref):
    deg = jnp.maximum(c0_ref[...] + c1_ref[...], 1.0)
    a = (p0_ref[...] + p1_ref[...]) / deg
    y = jnp.dot(a, w_ref[...], preferred_element_type=jnp.float32) + b_ref[...]
    o_ref[...] = jnp.maximum(y, 0.0) + h_ref[...]


def _layer(p0, p1, c0, c1, h, w, b):
    return pl.pallas_call(
        _layer_body,
        grid=(_N // _BN,),
        in_specs=[
            pl.BlockSpec((_BN, _H), lambda i: (i, 0)),
            pl.BlockSpec((_BN, _H), lambda i: (i, 0)),
            pl.BlockSpec((_BN, 1), lambda i: (i, 0)),
            pl.BlockSpec((_BN, 1), lambda i: (i, 0)),
            pl.BlockSpec((_BN, _H), lambda i: (i, 0)),
            pl.BlockSpec((_H, _H), lambda i: (0, 0)),
            pl.BlockSpec((1, _H), lambda i: (0, 0)),
        ],
        out_specs=pl.BlockSpec((_BN, _H), lambda i: (i, 0)),
        out_shape=jax.ShapeDtypeStruct((_N, _H), jnp.float32),
    )(p0, p1, c0, c1, h, w, b)


def _last_body(p0_ref, p1_ref, c0_ref, c1_ref, h_ref, w_ref, b_ref,
               wo_ref, bo_ref, o_ref):
    deg = jnp.maximum(c0_ref[...] + c1_ref[...], 1.0)
    a = (p0_ref[...] + p1_ref[...]) / deg
    t = jnp.dot(a, w_ref[...], preferred_element_type=jnp.float32) + b_ref[...]
    hh = jnp.maximum(t, 0.0) + h_ref[...]
    y = (jnp.dot(hh, wo_ref[...], preferred_element_type=jnp.float32)
         + bo_ref[...])
    m = jnp.max(y, axis=-1, keepdims=True)
    lse = jnp.log(jnp.sum(jnp.exp(y - m), axis=-1, keepdims=True)) + m
    o_ref[...] = y - lse


def _last(p0, p1, c0, c1, h, w, b, wo, bo):
    return pl.pallas_call(
        _last_body,
        grid=(_N // _BN,),
        in_specs=[
            pl.BlockSpec((_BN, _H), lambda i: (i, 0)),
            pl.BlockSpec((_BN, _H), lambda i: (i, 0)),
            pl.BlockSpec((_BN, 1), lambda i: (i, 0)),
            pl.BlockSpec((_BN, 1), lambda i: (i, 0)),
            pl.BlockSpec((_BN, _H), lambda i: (i, 0)),
            pl.BlockSpec((_H, _H), lambda i: (0, 0)),
            pl.BlockSpec((1, _H), lambda i: (0, 0)),
            pl.BlockSpec((_H, _C), lambda i: (0, 0)),
            pl.BlockSpec((1, _C), lambda i: (0, 0)),
        ],
        out_specs=pl.BlockSpec((_BN, _C), lambda i: (i, 0)),
        out_shape=jax.ShapeDtypeStruct((_N, _C), jnp.float32),
    )(p0, p1, c0, c1, h, w, b, wo, bo)


@jax.jit
def kernel(x, edge_index, W_embed, b_embed, conv_W, conv_b,
           bn_gamma, bn_beta, bn_mean, bn_var, W_out, b_out):
    ew = _E // _NW
    nmain = _NSB * _SB * _CH
    e0 = edge_index[0].reshape(_NW, ew)
    e1 = edge_index[1].reshape(_NW, ew)
    srcm = e0[:, :nmain].reshape(_NW, _NSB, _SB, _CH)
    dstm = e1[:, :nmain].reshape(_NW, _NSB, _SB, _CH)
    srct = e0[:, nmain:].reshape(_NW, 1, _TAIL)
    dstt = e1[:, nmain:].reshape(_NW, 1, _TAIL)
    zbig = jnp.zeros((_NP, _H), jnp.float32)
    zsmall = jnp.zeros((_NP,), jnp.float32)

    s = bn_gamma / jnp.sqrt(bn_var + 1e-5)
    w_fold = conv_W * s[:, None, :]
    b_fold = conv_b * s + bn_beta - bn_mean * s

    h = _embed(x, W_embed, b_embed.reshape(1, _H))

    c0 = c1 = None
    for i in range(3):
        if i == 0:
            agg, cnt = _sc_aggregate_cnt(h, srcm, dstm, srct, dstt, zbig, zsmall)
            c0 = cnt[0, :_N].reshape(_N, 1)
            c1 = cnt[1, :_N].reshape(_N, 1)
        else:
            (agg,) = _sc_aggregate(h, srcm, dstm, srct, dstt, zbig, zsmall)
        if i < 2:
            h = _layer(agg[0, :_N], agg[1, :_N], c0, c1, h,
                       w_fold[i], b_fold[i].reshape(1, _H))
        else:
            return _last(agg[0, :_N], agg[1, :_N], c0, c1, h,
                         w_fold[i], b_fold[i].reshape(1, _H),
                         W_out, b_out.reshape(1, _C))


# speedup vs baseline: 1.4670x; 1.1362x over previous
import functools

import jax
import jax.numpy as jnp
from jax import lax
from jax.experimental import pallas as pl
from jax.experimental.pallas import tpu as pltpu
from jax.experimental.pallas import tpu_sc as plsc

_N = 10000
_E = 320000
_H = 128
_C = 40

_CH = 80
_NW = 32
_SB = 25
_NSB = 5
_NP = 10240


def _sc_agg_body(with_cnt, h, srcm, dstm, zbig, zsmall,
                 *refs):
    if with_cnt:
        (agg_out, cnt_out,
         idx_s, idx_d, rows0, rows1, rows2, ones, agg_sh, cnt_sh,
         g0, g1, g2, s0, s1, s2) = refs
    else:
        (agg_out, idx_s, idx_d, rows0, rows1, rows2, agg_sh,
         g0, g1, g2, s0, s1, s2) = refs
    cid = lax.axis_index("c")
    sid = lax.axis_index("s")
    wid = cid * 16 + sid

    pltpu.sync_copy(zbig.at[pl.ds(sid * 640, 640)],
                    agg_sh.at[pl.ds(sid * 640, 640)])

    if with_cnt:
        pltpu.sync_copy(zsmall.at[pl.ds(sid * 640, 640)],
                        cnt_sh.at[pl.ds(sid * 640, 640)])
        for i in range(_CH // 16):
            ones[pl.ds(i * 16, 16)] = jnp.ones((16,), jnp.float32)

    plsc.subcore_barrier()

    def gather(c, rows, sem):
        pltpu.async_copy(h.at[idx_s.at[c]], rows, sem)

    def gwait(rows, sem):
        pltpu.make_async_copy(h.at[idx_s.at[0]], rows, sem).wait()

    def sc_issue(c, rows, sem):
        pltpu.async_copy(rows, agg_sh.at[idx_d.at[c]], sem, add=True)
        if with_cnt:
            pltpu.async_copy(ones, cnt_sh.at[idx_d.at[c]], sem, add=True)

    def swait(rows, sem):
        pltpu.make_async_copy(rows, agg_sh.at[idx_d.at[0]], sem).wait()
        if with_cnt:
            pltpu.make_async_copy(ones, cnt_sh.at[idx_d.at[0]], sem).wait()

    def superblock(sb, _):
        pltpu.sync_copy(srcm.at[wid].at[sb], idx_s)
        pltpu.sync_copy(dstm.at[wid].at[sb], idx_d)

        gather(0, rows0, g0)
        gather(1, rows1, g1)
        gwait(rows0, g0); sc_issue(0, rows0, s0); gather(2, rows2, g2)
        gwait(rows1, g1); sc_issue(1, rows1, s1)
        swait(rows0, s0); gather(3, rows0, g0)
        gwait(rows2, g2); sc_issue(2, rows2, s2)
        swait(rows1, s1); gather(4, rows1, g1)

        def triple(i, __):
            c = 3 * i
            gwait(rows0, g0); sc_issue(c, rows0, s0)
            swait(rows2, s2); gather(c + 2, rows2, g2)
            gwait(rows1, g1); sc_issue(c + 1, rows1, s1)
            swait(rows0, s0); gather(c + 3, rows0, g0)
            gwait(rows2, g2); sc_issue(c + 2, rows2, s2)
            swait(rows1, s1); gather(c + 4, rows1, g1)
            return ()

        lax.fori_loop(1, 7, triple, ())

        gwait(rows0, g0); sc_issue(21, rows0, s0)
        swait(rows2, s2); gather(23, rows2, g2)
        gwait(rows1, g1); sc_issue(22, rows1, s1)
        swait(rows0, s0); gather(24, rows0, g0)
        gwait(rows2, g2); sc_issue(23, rows2, s2)
        swait(rows1, s1)
        gwait(rows0, g0); sc_issue(24, rows0, s0)
        swait(rows2, s2)
        swait(rows0, s0)
        return ()

    lax.fori_loop(0, _NSB, superblock, ())

    plsc.subcore_barrier()

    pltpu.sync_copy(agg_sh.at[pl.ds(sid * 640, 640)],
                    agg_out.at[cid].at[pl.ds(sid * 640, 640)])

    if with_cnt:
        pltpu.sync_copy(cnt_sh.at[pl.ds(sid * 640, 640)],
                        cnt_out.at[cid].at[pl.ds(sid * 640, 640)])


def _make_sc_aggregate(with_cnt):
    out_type = [jax.ShapeDtypeStruct((2, _NP, _H), jnp.float32)]
    scratch = [
        pltpu.VMEM((_SB, _CH), jnp.int32),
        pltpu.VMEM((_SB, _CH), jnp.int32),
        pltpu.VMEM((_CH, _H), jnp.float32),
        pltpu.VMEM((_CH, _H), jnp.float32),
        pltpu.VMEM((_CH, _H), jnp.float32),
    ]
    if with_cnt:
        out_type.append(jax.ShapeDtypeStruct((2, _NP), jnp.float32))
        scratch.append(pltpu.VMEM((_CH,), jnp.float32))
    scratch.append(pltpu.VMEM_SHARED((_NP, _H), jnp.float32))
    if with_cnt:
        scratch.append(pltpu.VMEM_SHARED((_NP,), jnp.float32))
    scratch += [pltpu.SemaphoreType.DMA] * 6
    return pl.kernel(
        functools.partial(_sc_agg_body, with_cnt),
        out_type=out_type,
        mesh=plsc.VectorSubcoreMesh(core_axis_name="c", subcore_axis_name="s"),
        scratch_types=scratch,
    )


_sc_aggregate_cnt = _make_sc_aggregate(True)
_sc_aggregate = _make_sc_aggregate(False)


_BN = 1000


def _embed_body(x_ref, w_ref, b_ref, o_ref):
    o_ref[...] = (jnp.dot(x_ref[...], w_ref[...],
                          preferred_element_type=jnp.float32) + b_ref[...])


def _embed(x, w, b):
    return pl.pallas_call(
        _embed_body,
        grid=(_N // _BN,),
        in_specs=[
            pl.BlockSpec((_BN, _H), lambda i: (i, 0)),
            pl.BlockSpec((_H, _H), lambda i: (0, 0)),
            pl.BlockSpec((1, _H), lambda i: (0, 0)),
        ],
        out_specs=pl.BlockSpec((_BN, _H), lambda i: (i, 0)),
        out_shape=jax.ShapeDtypeStruct((_N, _H), jnp.float32),
    )(x, w, b)


def _layer_body(p_ref, c_ref, h_ref, w_ref, b_ref, o_ref):
    deg = jnp.maximum(c_ref[0] + c_ref[1], 1.0)
    a = (p_ref[0] + p_ref[1]) / deg
    y = jnp.dot(a, w_ref[...], preferred_element_type=jnp.float32) + b_ref[...]
    o_ref[...] = jnp.maximum(y, 0.0) + h_ref[...]


def _layer(p, c, h, w, b):
    return pl.pallas_call(
        _layer_body,
        grid=(_N // _BN,),
        in_specs=[
            pl.BlockSpec((2, _BN, _H), lambda i: (0, i, 0)),
            pl.BlockSpec((2, _BN, 1), lambda i: (0, i, 0)),
            pl.BlockSpec((_BN, _H), lambda i: (i, 0)),
            pl.BlockSpec((_H, _H), lambda i: (0, 0)),
            pl.BlockSpec((1, _H), lambda i: (0, 0)),
        ],
        out_specs=pl.BlockSpec((_BN, _H), lambda i: (i, 0)),
        out_shape=jax.ShapeDtypeStruct((_N, _H), jnp.float32),
    )(p, c, h, w, b)


def _last_body(p_ref, c_ref, h_ref, w_ref, b_ref,
               wo_ref, bo_ref, o_ref):
    deg = jnp.maximum(c_ref[0] + c_ref[1], 1.0)
    a = (p_ref[0] + p_ref[1]) / deg
    t = jnp.dot(a, w_ref[...], preferred_element_type=jnp.float32) + b_ref[...]
    hh = jnp.maximum(t, 0.0) + h_ref[...]
    y = (jnp.dot(hh, wo_ref[...], preferred_element_type=jnp.float32)
         + bo_ref[...])
    m = jnp.max(y, axis=-1, keepdims=True)
    lse = jnp.log(jnp.sum(jnp.exp(y - m), axis=-1, keepdims=True)) + m
    o_ref[...] = y - lse


def _last(p, c, h, w, b, wo, bo):
    return pl.pallas_call(
        _last_body,
        grid=(_N // _BN,),
        in_specs=[
            pl.BlockSpec((2, _BN, _H), lambda i: (0, i, 0)),
            pl.BlockSpec((2, _BN, 1), lambda i: (0, i, 0)),
            pl.BlockSpec((_BN, _H), lambda i: (i, 0)),
            pl.BlockSpec((_H, _H), lambda i: (0, 0)),
            pl.BlockSpec((1, _H), lambda i: (0, 0)),
            pl.BlockSpec((_H, _C), lambda i: (0, 0)),
            pl.BlockSpec((1, _C), lambda i: (0, 0)),
        ],
        out_specs=pl.BlockSpec((_BN, _C), lambda i: (i, 0)),
        out_shape=jax.ShapeDtypeStruct((_N, _C), jnp.float32),
    )(p, c, h, w, b, wo, bo)


@jax.jit
def kernel(x, edge_index, W_embed, b_embed, conv_W, conv_b,
           bn_gamma, bn_beta, bn_mean, bn_var, W_out, b_out):
    srcm = edge_index[0].reshape(_NW, _NSB, _SB, _CH)
    dstm = edge_index[1].reshape(_NW, _NSB, _SB, _CH)
    zbig = jnp.zeros((_NP, _H), jnp.float32)
    zsmall = jnp.zeros((_NP,), jnp.float32)

    s = bn_gamma / jnp.sqrt(bn_var + 1e-5)
    w_fold = conv_W * s[:, None, :]
    b_fold = conv_b * s + bn_beta - bn_mean * s

    h = _embed(x, W_embed, b_embed.reshape(1, _H))

    cnt3 = None
    for i in range(3):
        if i == 0:
            agg, cnt = _sc_aggregate_cnt(h, srcm, dstm, zbig, zsmall)
            cnt3 = cnt.reshape(2, _NP, 1)
        else:
            (agg,) = _sc_aggregate(h, srcm, dstm, zbig, zsmall)
        if i < 2:
            h = _layer(agg, cnt3, h, w_fold[i], b_fold[i].reshape(1, _H))
        else:
            return _last(agg, cnt3, h, w_fold[i], b_fold[i].reshape(1, _H),
                         W_out, b_out.reshape(1, _C))
